# baseline (speedup 1.0000x reference)
import jax
import jax.numpy as jnp
from jax import lax
from jax.experimental import pallas as pl
from jax.experimental.pallas import tpu as pltpu
from jax.experimental.pallas import tpu_sc as plsc

_NC = 2
_NS = 16
_NW = _NC * _NS
_TOPK = 8


def _topk_body(qn_ref, kn_ref, idx_ref):
    qn = qn_ref[...]
    kn = kn_ref[...]
    st = lax.dot_general(
        kn, qn, (((1,), (1,)), ((), ())),
        preferred_element_type=jnp.float32,
    )
    s = st.T
    b, p = s.shape
    n_rows = idx_ref.shape[1]
    e_len = (n_rows - _TOPK) // _TOPK
    g_len = n_rows - _TOPK * e_len
    iota = lax.broadcasted_iota(jnp.int32, (b, p), 1)
    sub_g = lax.broadcasted_iota(jnp.int32, (b, g_len), 1)
    sub = lax.broadcasted_iota(jnp.int32, (b, e_len), 1)
    idx_ref[:, pl.ds(0, g_len)] = p * e_len + sub_g
    cur = s
    for j in range(_TOPK):
        m = jnp.max(cur, axis=1, keepdims=True)
        sel = jnp.where(cur == m, iota, p)
        idx_j = jnp.min(sel, axis=1)
        idx_ref[:, pl.ds(g_len + j * e_len, e_len)] = idx_j[:, None] * e_len + sub
        cur = jnp.where(iota == idx_j[:, None], -jnp.inf, cur)


def _sc_gather_body(tab_hbm, idx_hbm, out_hbm, idx_v, buf0, buf1, sem0, sem1):
    rows_per = idx_v.shape[0]
    base = (lax.axis_index("s") * _NC + lax.axis_index("c")) * rows_per

    pltpu.sync_copy(idx_hbm.at[pl.ds(base, rows_per)], idx_v)

    def start(i, buf, sem):
        pltpu.async_copy(tab_hbm.at[idx_v.at[i]], buf, sem)

    def wait(i, buf, sem):
        pltpu.make_async_copy(tab_hbm.at[idx_v.at[i]], buf, sem).wait()

    start(0, buf0, sem0)
    def body(j, carry):
        i = 2 * j
        wait(i, buf0, sem0)
        start(i + 1, buf1, sem1)
        pltpu.sync_copy(buf0, out_hbm.at[base + i])
        wait(i + 1, buf1, sem1)
        @pl.when(i + 2 < rows_per)
        def _():
            start(i + 2, buf0, sem0)
        pltpu.sync_copy(buf1, out_hbm.at[base + i + 1])
        return carry
    lax.fori_loop(0, rows_per // 2, body, 0)


def kernel(query, g_prompt, e_prompt_pool, e_prompt_keys):
    b, d = query.shape
    pool, e_len, _ = e_prompt_pool.shape
    g_len = g_prompt.shape[1]
    n_rows = g_len + _TOPK * e_len

    qn = query / jnp.linalg.norm(query, axis=-1, keepdims=True)
    kn = e_prompt_keys / jnp.linalg.norm(e_prompt_keys, axis=-1, keepdims=True)

    idx = pl.pallas_call(
        _topk_body,
        out_shape=jax.ShapeDtypeStruct((b, n_rows), jnp.int32),
    )(qn, kn)

    table = jnp.concatenate(
        [e_prompt_pool.reshape(pool * e_len, d), g_prompt.reshape(g_len, d)])

    rows_per = b // _NW
    sc = pl.kernel(
        _sc_gather_body,
        out_type=jax.ShapeDtypeStruct((b, n_rows, d), jnp.float32),
        mesh=plsc.VectorSubcoreMesh(core_axis_name="c", subcore_axis_name="s"),
        scratch_types=[
            pltpu.VMEM((rows_per, n_rows), jnp.int32),
            pltpu.VMEM((n_rows, d), jnp.float32),
            pltpu.VMEM((n_rows, d), jnp.float32),
            pltpu.SemaphoreType.DMA,
            pltpu.SemaphoreType.DMA,
        ],
    )
    return sc(table, idx)


# speedup vs baseline: 1.2815x; 1.2815x over previous
import jax
import jax.numpy as jnp
from jax import lax
from jax.experimental import pallas as pl
from jax.experimental.pallas import tpu as pltpu
from jax.experimental.pallas import tpu_sc as plsc

_NC = 2
_NS = 16
_NW = _NC * _NS
_TOPK = 8


def _topk_body(qn_ref, kn_ref, idx_ref):
    qn = qn_ref[...]
    kn = kn_ref[...]
    st = lax.dot_general(
        kn, qn, (((1,), (1,)), ((), ())),
        preferred_element_type=jnp.float32,
    )
    s = st.T
    b, p = s.shape
    n_e = idx_ref.shape[1]
    e_len = n_e // _TOPK
    iota = lax.broadcasted_iota(jnp.int32, (b, p), 1)
    sub = lax.broadcasted_iota(jnp.int32, (b, e_len), 1)
    cur = s
    for j in range(_TOPK):
        m = jnp.max(cur, axis=1, keepdims=True)
        sel = jnp.where(cur == m, iota, p)
        idx_j = jnp.min(sel, axis=1)
        idx_ref[:, pl.ds(j * e_len, e_len)] = idx_j[:, None] * e_len + sub
        cur = jnp.where(iota == idx_j[:, None], -jnp.inf, cur)


def _sc_gather_body(pool_hbm, g_hbm, idx_hbm, out_hbm, idx_v, buf0, buf1,
                    sem0, sem1):
    rows_per = idx_v.shape[0]
    g_len = g_hbm.shape[0]
    n_e = idx_v.shape[1]
    base = (lax.axis_index("s") * _NC + lax.axis_index("c")) * rows_per

    pltpu.sync_copy(idx_hbm.at[pl.ds(base, rows_per)], idx_v)
    pltpu.sync_copy(g_hbm, buf0.at[pl.ds(0, g_len)])
    pltpu.sync_copy(g_hbm, buf1.at[pl.ds(0, g_len)])

    def start(i, buf, sem):
        pltpu.async_copy(pool_hbm.at[idx_v.at[i]], buf.at[pl.ds(g_len, n_e)],
                         sem)

    def wait(i, buf, sem):
        pltpu.make_async_copy(pool_hbm.at[idx_v.at[i]],
                              buf.at[pl.ds(g_len, n_e)], sem).wait()

    start(0, buf0, sem0)
    def body(j, carry):
        i = 2 * j
        wait(i, buf0, sem0)
        start(i + 1, buf1, sem1)
        pltpu.sync_copy(buf0, out_hbm.at[base + i])
        wait(i + 1, buf1, sem1)
        @pl.when(i + 2 < rows_per)
        def _():
            start(i + 2, buf0, sem0)
        pltpu.sync_copy(buf1, out_hbm.at[base + i + 1])
        return carry
    lax.fori_loop(0, rows_per // 2, body, 0)


def kernel(query, g_prompt, e_prompt_pool, e_prompt_keys):
    b, d = query.shape
    pool, e_len, _ = e_prompt_pool.shape
    g_len = g_prompt.shape[1]
    n_e = _TOPK * e_len

    qn = query / jnp.linalg.norm(query, axis=-1, keepdims=True)
    kn = e_prompt_keys / jnp.linalg.norm(e_prompt_keys, axis=-1, keepdims=True)

    idx = pl.pallas_call(
        _topk_body,
        out_shape=jax.ShapeDtypeStruct((b, n_e), jnp.int32),
    )(qn, kn)

    pool_rows = e_prompt_pool.reshape(pool * e_len, d)
    g_rows = g_prompt.reshape(g_len, d)

    rows_per = b // _NW
    sc = pl.kernel(
        _sc_gather_body,
        out_type=jax.ShapeDtypeStruct((b, g_len + n_e, d), jnp.float32),
        mesh=plsc.VectorSubcoreMesh(core_axis_name="c", subcore_axis_name="s"),
        scratch_types=[
            pltpu.VMEM((rows_per, n_e), jnp.int32),
            pltpu.VMEM((g_len + n_e, d), jnp.float32),
            pltpu.VMEM((g_len + n_e, d), jnp.float32),
            pltpu.SemaphoreType.DMA,
            pltpu.SemaphoreType.DMA,
        ],
    )
    return sc(pool_rows, g_rows, idx)


# speedup vs baseline: 1.3527x; 1.0556x over previous
import jax
import jax.numpy as jnp
from jax import lax
from jax.experimental import pallas as pl
from jax.experimental.pallas import tpu as pltpu
from jax.experimental.pallas import tpu_sc as plsc

_NC = 2
_NS = 16
_NW = _NC * _NS
_TOPK = 8
_REP = 8


def _topk_body(qn_ref, kn_ref, idx_ref):
    qn = qn_ref[...]
    kn = kn_ref[...]
    st = lax.dot_general(
        kn, qn, (((1,), (1,)), ((), ())),
        preferred_element_type=jnp.float32,
    )
    s = st.T
    b, p = s.shape
    n_e = idx_ref.shape[1]
    e_len = n_e // _TOPK
    rows_per = b // _NW
    iota = lax.broadcasted_iota(jnp.int32, (b, p), 1)
    sub = lax.broadcasted_iota(jnp.int32, (b, e_len), 1)
    row = lax.broadcasted_iota(jnp.int32, (b, 1), 0)
    rep_off = ((row // rows_per) % _REP) * (p * e_len)
    cur = s
    for j in range(_TOPK):
        m = jnp.max(cur, axis=1, keepdims=True)
        sel = jnp.where(cur == m, iota, p)
        idx_j = jnp.min(sel, axis=1)
        idx_ref[:, pl.ds(j * e_len, e_len)] = (idx_j[:, None] * e_len + sub
                                               + rep_off)
        cur = jnp.where(iota == idx_j[:, None], -jnp.inf, cur)


def _sc_gather_body(pool_hbm, g_hbm, idx_hbm, out_hbm, idx_v, buf0, buf1,
                    sem0, sem1):
    rows_per = idx_v.shape[0]
    g_len = g_hbm.shape[0]
    n_e = idx_v.shape[1]
    base = (lax.axis_index("s") * _NC + lax.axis_index("c")) * rows_per

    pltpu.sync_copy(idx_hbm.at[pl.ds(base, rows_per)], idx_v)
    pltpu.sync_copy(g_hbm, buf0.at[pl.ds(0, g_len)])
    pltpu.sync_copy(g_hbm, buf1.at[pl.ds(0, g_len)])

    def start(i, buf, sem):
        pltpu.async_copy(pool_hbm.at[idx_v.at[i]], buf.at[pl.ds(g_len, n_e)],
                         sem)

    def wait(i, buf, sem):
        pltpu.make_async_copy(pool_hbm.at[idx_v.at[i]],
                              buf.at[pl.ds(g_len, n_e)], sem).wait()

    start(0, buf0, sem0)
    def body(j, carry):
        i = 2 * j
        wait(i, buf0, sem0)
        start(i + 1, buf1, sem1)
        pltpu.sync_copy(buf0, out_hbm.at[base + i])
        wait(i + 1, buf1, sem1)
        @pl.when(i + 2 < rows_per)
        def _():
            start(i + 2, buf0, sem0)
        pltpu.sync_copy(buf1, out_hbm.at[base + i + 1])
        return carry
    lax.fori_loop(0, rows_per // 2, body, 0)


def kernel(query, g_prompt, e_prompt_pool, e_prompt_keys):
    b, d = query.shape
    pool, e_len, _ = e_prompt_pool.shape
    g_len = g_prompt.shape[1]
    n_e = _TOPK * e_len

    qn = query / jnp.linalg.norm(query, axis=-1, keepdims=True)
    kn = e_prompt_keys / jnp.linalg.norm(e_prompt_keys, axis=-1, keepdims=True)

    idx = pl.pallas_call(
        _topk_body,
        out_shape=jax.ShapeDtypeStruct((b, n_e), jnp.int32),
    )(qn, kn)

    pool_rows = jnp.tile(e_prompt_pool.reshape(pool * e_len, d), (_REP, 1))
    g_rows = g_prompt.reshape(g_len, d)

    rows_per = b // _NW
    sc = pl.kernel(
        _sc_gather_body,
        out_type=jax.ShapeDtypeStruct((b, g_len + n_e, d), jnp.float32),
        mesh=plsc.VectorSubcoreMesh(core_axis_name="c", subcore_axis_name="s"),
        scratch_types=[
            pltpu.VMEM((rows_per, n_e), jnp.int32),
            pltpu.VMEM((g_len + n_e, d), jnp.float32),
            pltpu.VMEM((g_len + n_e, d), jnp.float32),
            pltpu.SemaphoreType.DMA,
            pltpu.SemaphoreType.DMA,
        ],
    )
    return sc(pool_rows, g_rows, idx)


# speedup vs baseline: 1.3559x; 1.0024x over previous
import jax
import jax.numpy as jnp
from jax import lax
from jax.experimental import pallas as pl
from jax.experimental.pallas import tpu as pltpu
from jax.experimental.pallas import tpu_sc as plsc

_NC = 2
_NS = 16
_NW = _NC * _NS
_TOPK = 8
_REP = 8


def _topk_body(qn_ref, kn_ref, idx_ref):
    qn = qn_ref[...]
    kn = kn_ref[...]
    st = lax.dot_general(
        kn, qn, (((1,), (1,)), ((), ())),
        preferred_element_type=jnp.float32,
    )
    s = st.T
    b, p = s.shape
    n_e = idx_ref.shape[1]
    e_len = n_e // _TOPK
    rows_per = b // _NW
    iota = lax.broadcasted_iota(jnp.int32, (b, p), 1)
    sub = lax.broadcasted_iota(jnp.int32, (b, e_len), 1)
    row = lax.broadcasted_iota(jnp.int32, (b, 1), 0)
    rep_off = ((row // rows_per) % _REP) * (p * e_len)
    cur = s
    for j in range(_TOPK):
        m = jnp.max(cur, axis=1, keepdims=True)
        sel = jnp.where(cur == m, iota, p)
        idx_j = jnp.min(sel, axis=1)
        idx_ref[:, pl.ds(j * e_len, e_len)] = (idx_j[:, None] * e_len + sub
                                               + rep_off)
        cur = jnp.where(iota == idx_j[:, None], -jnp.inf, cur)


def _sc_gather_body(pool_hbm, g_hbm, idx_hbm, out_hbm, idx_v, buf0, buf1,
                    sg0, sg1, sw0, sw1):
    rows_per = idx_v.shape[0]
    g_len = g_hbm.shape[0]
    n_e = idx_v.shape[1]
    base = (lax.axis_index("s") * _NC + lax.axis_index("c")) * rows_per

    pltpu.sync_copy(idx_hbm.at[pl.ds(base, rows_per)], idx_v)
    pltpu.sync_copy(g_hbm, buf0.at[pl.ds(0, g_len)])
    pltpu.sync_copy(g_hbm, buf1.at[pl.ds(0, g_len)])

    slots = ((buf0, sg0, sw0), (buf1, sg1, sw1))

    def g_copy(i, slot):
        buf, sg, _ = slots[slot]
        return pltpu.make_async_copy(pool_hbm.at[idx_v.at[i]],
                                     buf.at[pl.ds(g_len, n_e)], sg)

    def w_copy(i, slot):
        buf, _, sw = slots[slot]
        return pltpu.make_async_copy(buf, out_hbm.at[base + i], sw)

    g_copy(0, 0).start()

    def body(k, carry):
        for j in range(2):
            i = 2 * k + j
            s, s_o = j, 1 - j
            g_copy(i, s).wait()
            @pl.when(jnp.logical_and(i >= 1, i + 1 < rows_per))
            def _(i=i, s_o=s_o):
                w_copy(i - 1, s_o).wait()
            @pl.when(i + 1 < rows_per)
            def _(i=i, s_o=s_o):
                g_copy(i + 1, s_o).start()
            w_copy(i, s).start()
        return carry
    lax.fori_loop(0, rows_per // 2, body, 0)
    w_copy(rows_per - 2, 0).wait()
    w_copy(rows_per - 1, 1).wait()


def kernel(query, g_prompt, e_prompt_pool, e_prompt_keys):
    b, d = query.shape
    pool, e_len, _ = e_prompt_pool.shape
    g_len = g_prompt.shape[1]
    n_e = _TOPK * e_len

    qn = query / jnp.linalg.norm(query, axis=-1, keepdims=True)
    kn = e_prompt_keys / jnp.linalg.norm(e_prompt_keys, axis=-1, keepdims=True)

    idx = pl.pallas_call(
        _topk_body,
        out_shape=jax.ShapeDtypeStruct((b, n_e), jnp.int32),
    )(qn, kn)

    pool_rows = jnp.tile(e_prompt_pool.reshape(pool * e_len, d), (_REP, 1))
    g_rows = g_prompt.reshape(g_len, d)

    rows_per = b // _NW
    sc = pl.kernel(
        _sc_gather_body,
        out_type=jax.ShapeDtypeStruct((b, g_len + n_e, d), jnp.float32),
        mesh=plsc.VectorSubcoreMesh(core_axis_name="c", subcore_axis_name="s"),
        scratch_types=[
            pltpu.VMEM((rows_per, n_e), jnp.int32),
            pltpu.VMEM((g_len + n_e, d), jnp.float32),
            pltpu.VMEM((g_len + n_e, d), jnp.float32),
            pltpu.SemaphoreType.DMA,
            pltpu.SemaphoreType.DMA,
            pltpu.SemaphoreType.DMA,
            pltpu.SemaphoreType.DMA,
        ],
    )
    return sc(pool_rows, g_rows, idx)
